# Initial kernel scaffold; baseline (speedup 1.0000x reference)
import jax
import jax.numpy as jnp
from jax.experimental import pallas as pl


def kernel(node_features, edge_features, Esrc, Etgt, batch, W_in, b_in, ee_W1, ee_b1, ee_W2, ee_b2, Wz, Uz, bz, Wr, Ur, br, Wn, Un, bn, W_out, b_out):
    raise NotImplementedError("write your pallas kernel here")


# speedup vs baseline: 1.2220x; 1.2220x over previous
import functools

import jax
import jax.numpy as jnp
from jax import lax
from jax.experimental import pallas as pl
from jax.experimental.pallas import tpu as pltpu
from jax.experimental.pallas import tpu_sc as plsc

F32 = jnp.float32
HIGH = lax.Precision.HIGHEST

H = 16
HH = H * H
NC = 2
NS = 16
NW = NC * NS
CH = 128


def _dot(a, b):
    return jnp.dot(a, b, preferred_element_type=F32, precision=HIGH)


def _proj_body(nf, w, b, out):
    out[...] = _dot(nf[...], w[...]) + b[...]


def _msgs_body(ef, hs, w1, b1, w2, b2, r, s, out):
    eh = jnp.maximum(_dot(ef[...], w1[...]) + b1[...], 0.0)
    a = _dot(eh, w2[...]) + b2[...]
    hrep = _dot(hs[...], r[...])
    out[...] = _dot(a * hrep, s[...])


def _gru_core(ma, mb, h, wz, uz, bz, wr, ur, br, wn, un, bn):
    m = ma + mb
    z = jax.nn.sigmoid(_dot(m, wz) + _dot(h, uz) + bz)
    r = jax.nn.sigmoid(_dot(m, wr) + _dot(h, ur) + br)
    n = jnp.tanh(_dot(m, wn) + r * _dot(h, un) + bn)
    return (1.0 - z) * n + z * h


def _gru_body(ma, mb, h, wz, uz, bz, wr, ur, br, wn, un, bn, out):
    out[...] = _gru_core(ma[0], mb[0], h[...], wz[...], uz[...], bz[...],
                         wr[...], ur[...], br[...], wn[...], un[...], bn[...])


def _gru_readout_body(ma, mb, h, wz, uz, bz, wr, ur, br, wn, un, bn,
                      bt, wo, bo, hout, gout, acc):
    i = pl.program_id(0)
    nb = pl.num_programs(0)
    hn = _gru_core(ma[0], mb[0], h[...], wz[...], uz[...], bz[...],
                   wr[...], ur[...], br[...], wn[...], un[...], bn[...])
    hout[...] = hn
    o = _dot(hn, wo[...]) + bo[...]
    ng = acc.shape[0]
    tn = o.shape[0]
    rows = lax.broadcasted_iota(jnp.int32, (ng, tn), 0)
    onehot = (bt[0] == rows).astype(F32)
    contrib = _dot(onehot, o)

    @pl.when(i == 0)
    def _():
        acc[...] = jnp.zeros_like(acc)

    acc[...] += contrib

    @pl.when(i == nb - 1)
    def _():
        gout[...] = acc[...]


def _make_sc_gather(n_nodes, e_pad, epw, nchunk):
    mesh = plsc.VectorSubcoreMesh(core_axis_name="c", subcore_axis_name="s",
                                  num_cores=NC, num_subcores=NS)

    @functools.partial(
        pl.kernel,
        mesh=mesh,
        out_type=jax.ShapeDtypeStruct((e_pad, H), F32),
        scratch_types=[
            pltpu.VMEM((nchunk, CH), jnp.int32),
            pltpu.VMEM((epw, H), F32),
            pltpu.SemaphoreType.DMA,
        ],
        compiler_params=pltpu.CompilerParams(use_tc_tiling_on_sc=False),
    )
    def sc_gather(h_hbm, idx_hbm, out_hbm, idx_v, rows_v, sem):
        c = lax.axis_index("c")
        s = lax.axis_index("s")
        wid = s * NC + c
        pltpu.sync_copy(idx_hbm.at[wid], idx_v)

        def body(j, carry):
            pltpu.async_copy(
                h_hbm.at[idx_v.at[j]],
                rows_v.at[pl.ds(j * CH, CH), :],
                sem,
            ).wait()
            return carry

        lax.fori_loop(0, nchunk, body, 0)
        pltpu.sync_copy(rows_v, out_hbm.at[pl.ds(wid * epw, epw), :])

    return sc_gather


def _make_sc_scatter(m_pad, e_pad, epw, nchunk):
    mesh = plsc.VectorSubcoreMesh(core_axis_name="c", subcore_axis_name="s",
                                  num_cores=NC, num_subcores=NS)
    rpt = m_pad // NS

    @functools.partial(
        pl.kernel,
        mesh=mesh,
        out_type=jax.ShapeDtypeStruct((NC, m_pad, H), F32),
        scratch_types=[
            pltpu.VMEM((nchunk, CH), jnp.int32),
            pltpu.VMEM((epw, H), F32),
            pltpu.VMEM_SHARED((m_pad, H), F32),
        ],
        compiler_params=pltpu.CompilerParams(use_tc_tiling_on_sc=False),
    )
    def sc_scatter(msgs_hbm, idx_hbm, zeros_hbm, out_hbm, idx_v, msg_v, macc):
        c = lax.axis_index("c")
        s = lax.axis_index("s")
        wid = s * NC + c

        @pl.when(s == 0)
        def _():
            pltpu.sync_copy(zeros_hbm, macc)

        pltpu.sync_copy(idx_hbm.at[wid], idx_v)
        pltpu.sync_copy(msgs_hbm.at[pl.ds(wid * epw, epw), :], msg_v)
        plsc.subcore_barrier()

        def body(j, carry):
            pltpu.sync_copy(
                msg_v.at[pl.ds(j * CH, CH), :],
                macc.at[idx_v.at[j]],
                add=True,
            )
            return carry

        lax.fori_loop(0, nchunk, body, 0)
        plsc.subcore_barrier()

        pltpu.sync_copy(macc.at[pl.ds(s * rpt, rpt), :],
                        msg_v.at[pl.ds(0, rpt), :])
        pltpu.sync_copy(msg_v.at[pl.ds(0, rpt), :],
                        out_hbm.at[c, pl.ds(s * rpt, rpt), :])

    return sc_scatter


def kernel(node_features, edge_features, Esrc, Etgt, batch, W_in, b_in,
           ee_W1, ee_b1, ee_W2, ee_b2, Wz, Uz, bz, Wr, Ur, br, Wn, Un, bn,
           W_out, b_out):
    n, f = node_features.shape
    e, de = edge_features.shape
    ng = 64
    t_rounds = 3

    e_pad = ((e + NW * CH - 1) // (NW * CH)) * (NW * CH)
    epw = e_pad // NW
    nchunk = epw // CH
    m_pad = ((n + 1 + NS * 8 - 1) // (NS * 8)) * (NS * 8)
    tn = 2000
    nb = n // tn
    te = 2048
    neb = e_pad // te

    ef_pad = jnp.pad(edge_features, ((0, e_pad - e), (0, 0)))
    esrc_p = jnp.pad(Esrc, (0, e_pad - e)).reshape(NW, nchunk, CH)
    etgt_p = jnp.pad(Etgt, (0, e_pad - e), constant_values=n).reshape(
        NW, nchunk, CH)
    zeros_m = jnp.zeros((m_pad, H), F32)
    batch3 = batch.reshape(nb, 1, tn)

    rep = jnp.equal(jnp.arange(HH)[None, :] % H,
                    jnp.arange(H)[:, None]).astype(F32)
    sel = jnp.equal(jnp.arange(HH)[:, None] // H,
                    jnp.arange(H)[None, :]).astype(F32)

    b_in2 = b_in.reshape(1, H)
    ee_b1_2 = ee_b1.reshape(1, H)
    ee_b2_2 = ee_b2.reshape(1, HH)
    bz2 = bz.reshape(1, H)
    br2 = br.reshape(1, H)
    bn2 = bn.reshape(1, H)
    b_out2 = b_out.reshape(1, 1)

    full = lambda shape: pl.BlockSpec(shape, lambda i: tuple(0 for _ in shape))
    h = pl.pallas_call(
        _proj_body,
        grid=(nb,),
        in_specs=[
            pl.BlockSpec((tn, f), lambda i: (i, 0)),
            full((f, H)),
            full((1, H)),
        ],
        out_specs=pl.BlockSpec((tn, H), lambda i: (i, 0)),
        out_shape=jax.ShapeDtypeStruct((n, H), F32),
    )(node_features, W_in, b_in2)

    sc_gather = _make_sc_gather(n, e_pad, epw, nchunk)
    sc_scatter = _make_sc_scatter(m_pad, e_pad, epw, nchunk)

    msgs_call = pl.pallas_call(
        _msgs_body,
        grid=(neb,),
        in_specs=[
            pl.BlockSpec((te, de), lambda i: (i, 0)),
            pl.BlockSpec((te, H), lambda i: (i, 0)),
            full((de, H)),
            full((1, H)),
            full((H, HH)),
            full((1, HH)),
            full((H, HH)),
            full((HH, H)),
        ],
        out_specs=pl.BlockSpec((te, H), lambda i: (i, 0)),
        out_shape=jax.ShapeDtypeStruct((e_pad, H), F32),
    )

    gru_in_specs = [
        pl.BlockSpec((1, tn, H), lambda i: (0, i, 0)),
        pl.BlockSpec((1, tn, H), lambda i: (1, i, 0)),
        pl.BlockSpec((tn, H), lambda i: (i, 0)),
        full((H, H)), full((H, H)), full((1, H)),
        full((H, H)), full((H, H)), full((1, H)),
        full((H, H)), full((H, H)), full((1, H)),
    ]
    gru_call = pl.pallas_call(
        _gru_body,
        grid=(nb,),
        in_specs=gru_in_specs,
        out_specs=pl.BlockSpec((tn, H), lambda i: (i, 0)),
        out_shape=jax.ShapeDtypeStruct((n, H), F32),
    )
    gru_readout_call = pl.pallas_call(
        _gru_readout_body,
        grid=(nb,),
        in_specs=gru_in_specs + [
            pl.BlockSpec((1, 1, tn), lambda i: (i, 0, 0)),
            full((H, 1)),
            full((1, 1)),
        ],
        out_specs=[
            pl.BlockSpec((tn, H), lambda i: (i, 0)),
            pl.BlockSpec((ng, 1), lambda i: (0, 0)),
        ],
        out_shape=[
            jax.ShapeDtypeStruct((n, H), F32),
            jax.ShapeDtypeStruct((ng, 1), F32),
        ],
        scratch_shapes=[pltpu.VMEM((ng, 1), F32)],
    )

    g = None
    for t in range(t_rounds):
        hs = sc_gather(h, esrc_p)
        msgs = msgs_call(ef_pad, hs, ee_W1, ee_b1_2, ee_W2, ee_b2_2, rep, sel)
        m2 = sc_scatter(msgs, etgt_p, zeros_m)
        gru_args = (m2, m2, h, Wz, Uz, bz2, Wr, Ur, br2, Wn, Un, bn2)
        if t < t_rounds - 1:
            h = gru_call(*gru_args)
        else:
            h, g = gru_readout_call(*gru_args, batch3, W_out, b_out2)
    return g


# speedup vs baseline: 1.7416x; 1.4253x over previous
import functools

import jax
import jax.numpy as jnp
from jax import lax
from jax.experimental import pallas as pl
from jax.experimental.pallas import tpu as pltpu
from jax.experimental.pallas import tpu_sc as plsc

F32 = jnp.float32
HIGH = lax.Precision.HIGHEST

H = 16
HH = H * H
PK = 8
NC = 2
NS = 16
NW = NC * NS
CH = 128
NG = 64


def _dot(a, b):
    return jnp.dot(a, b, preferred_element_type=F32, precision=HIGH)


def _kron8(w):
    return jnp.kron(jnp.eye(PK, dtype=w.dtype), w)


def _tile8(b):
    return jnp.tile(b.reshape(1, -1), (1, PK)).reshape(1, -1)


def _proj_body(nf, w, b, out):
    out[...] = _dot(nf[...], w[...]) + b[...]


def _msgs_body(ef, hs, w1b, b1t, w2b, b2t, rb, sb, out):
    eh = jnp.maximum(_dot(ef[...], w1b[...]) + b1t[...], 0.0)
    a = _dot(eh, w2b[...]) + b2t[...]
    hrep = _dot(hs[...], rb[...])
    out[...] = _dot(a * hrep, sb[...])


def _gru_core(ma, mb, h, wz, uz, bz, wr, ur, br, wn, un, bn):
    m = ma + mb
    z = jax.nn.sigmoid(_dot(m, wz) + _dot(h, uz) + bz)
    r = jax.nn.sigmoid(_dot(m, wr) + _dot(h, ur) + br)
    n = jnp.tanh(_dot(m, wn) + r * _dot(h, un) + bn)
    return (1.0 - z) * n + z * h


def _gru_body(ma, mb, h, wz, uz, bz, wr, ur, br, wn, un, bn, out):
    np_ = h.shape[0]
    out[...] = _gru_core(ma[0, :np_], mb[0, :np_], h[...],
                         wz[...], uz[...], bz[...],
                         wr[...], ur[...], br[...], wn[...], un[...], bn[...])


def _gru_out_body(ma, mb, h, wz, uz, bz, wr, ur, br, wn, un, bn, wob, oout):
    np_ = h.shape[0]
    hn = _gru_core(ma[0, :np_], mb[0, :np_], h[...],
                   wz[...], uz[...], bz[...],
                   wr[...], ur[...], br[...], wn[...], un[...], bn[...])
    oout[...] = _dot(hn, wob[...])


def _readout_body(o3, b3, bo, g, acc):
    i = pl.program_id(0)
    nb = pl.num_programs(0)
    tn = o3.shape[2]
    rows = lax.broadcasted_iota(jnp.int32, (NG, tn), 0)
    oh = (b3[0] == rows).astype(F32)
    contrib = jnp.sum(oh * o3[0], axis=1, keepdims=True)
    cnt = jnp.sum(oh, axis=1, keepdims=True)

    @pl.when(i == 0)
    def _():
        acc[...] = jnp.zeros_like(acc)

    acc[...] += contrib + cnt * bo[0, 0]

    @pl.when(i == nb - 1)
    def _():
        g[...] = acc[...]


def _make_sc_gather(n_nodes, e_pad, epw, nchunk):
    mesh = plsc.VectorSubcoreMesh(core_axis_name="c", subcore_axis_name="s",
                                  num_cores=NC, num_subcores=NS)

    @functools.partial(
        pl.kernel,
        mesh=mesh,
        out_type=jax.ShapeDtypeStruct((e_pad, H), F32),
        scratch_types=[
            pltpu.VMEM((nchunk, CH), jnp.int32),
            pltpu.VMEM((epw, H), F32),
            pltpu.SemaphoreType.DMA,
        ],
        compiler_params=pltpu.CompilerParams(use_tc_tiling_on_sc=False),
    )
    def sc_gather(h_hbm, idx_hbm, out_hbm, idx_v, rows_v, sem):
        c = lax.axis_index("c")
        s = lax.axis_index("s")
        wid = s * NC + c
        pltpu.sync_copy(idx_hbm.at[wid], idx_v)

        def body(j, carry):
            pltpu.async_copy(
                h_hbm.at[idx_v.at[j]],
                rows_v.at[pl.ds(j * CH, CH), :],
                sem,
            ).wait()
            return carry

        lax.fori_loop(0, nchunk, body, 0)
        pltpu.sync_copy(rows_v, out_hbm.at[pl.ds(wid * epw, epw), :])

    return sc_gather


def _make_sc_scatter(m_pad, e_pad, epw, nchunk):
    mesh = plsc.VectorSubcoreMesh(core_axis_name="c", subcore_axis_name="s",
                                  num_cores=NC, num_subcores=NS)
    rpt = m_pad // NS

    @functools.partial(
        pl.kernel,
        mesh=mesh,
        out_type=jax.ShapeDtypeStruct((NC, m_pad, H), F32),
        scratch_types=[
            pltpu.VMEM((nchunk, CH), jnp.int32),
            pltpu.VMEM((epw, H), F32),
            pltpu.VMEM_SHARED((m_pad, H), F32),
        ],
        compiler_params=pltpu.CompilerParams(use_tc_tiling_on_sc=False),
    )
    def sc_scatter(msgs_hbm, idx_hbm, zeros_hbm, out_hbm, idx_v, msg_v, macc):
        c = lax.axis_index("c")
        s = lax.axis_index("s")
        wid = s * NC + c

        @pl.when(s == 0)
        def _():
            pltpu.sync_copy(zeros_hbm, macc)

        pltpu.sync_copy(idx_hbm.at[wid], idx_v)
        pltpu.sync_copy(msgs_hbm.at[pl.ds(wid * epw, epw), :], msg_v)
        plsc.subcore_barrier()

        def body(j, carry):
            pltpu.sync_copy(
                msg_v.at[pl.ds(j * CH, CH), :],
                macc.at[idx_v.at[j]],
                add=True,
            )
            return carry

        lax.fori_loop(0, nchunk, body, 0)
        plsc.subcore_barrier()

        pltpu.sync_copy(macc.at[pl.ds(s * rpt, rpt), :],
                        msg_v.at[pl.ds(0, rpt), :])
        pltpu.sync_copy(msg_v.at[pl.ds(0, rpt), :],
                        out_hbm.at[c, pl.ds(s * rpt, rpt), :])

    return sc_scatter


def kernel(node_features, edge_features, Esrc, Etgt, batch, W_in, b_in,
           ee_W1, ee_b1, ee_W2, ee_b2, Wz, Uz, bz, Wr, Ur, br, Wn, Un, bn,
           W_out, b_out):
    n, f = node_features.shape
    e, de = edge_features.shape
    t_rounds = 3

    e_pad = ((e + NW * CH - 1) // (NW * CH)) * (NW * CH)
    epw = e_pad // NW
    nchunk = epw // CH
    m_pad = ((n + 1 + NS * PK - 1) // (NS * PK)) * (NS * PK)
    np_ = n // PK
    mp_ = m_pad // PK
    ep_ = e_pad // PK
    tn = 2000
    nb = n // tn
    tnp = np_ // nb
    tep = 256
    neb = ep_ // tep

    ef_p = jnp.pad(edge_features.reshape(e // PK, PK * de),
                   ((0, ep_ - e // PK), (0, 0)))
    esrc_p = jnp.pad(Esrc, (0, e_pad - e)).reshape(NW, nchunk, CH)
    etgt_p = jnp.pad(Etgt, (0, e_pad - e), constant_values=n).reshape(
        NW, nchunk, CH)
    zeros_m = jnp.zeros((m_pad, H), F32)
    batch3 = batch.reshape(nb, 1, tn)
    nf_p = node_features.reshape(np_, PK * f)

    rep = jnp.equal(jnp.arange(HH)[None, :] % H,
                    jnp.arange(H)[:, None]).astype(F32)
    sel = jnp.equal(jnp.arange(HH)[:, None] // H,
                    jnp.arange(H)[None, :]).astype(F32)

    w_inb = _kron8(W_in)
    w1b = _kron8(ee_W1)
    w2b = _kron8(ee_W2)
    rb = _kron8(rep)
    sb = _kron8(sel)
    uzb, urb, unb = _kron8(Uz), _kron8(Ur), _kron8(Un)
    wzb, wrb, wnb = _kron8(Wz), _kron8(Wr), _kron8(Wn)
    wob = _kron8(W_out)
    b_int = _tile8(b_in)
    b1t = _tile8(ee_b1)
    b2t = _tile8(ee_b2)
    bzt, brt, bnt = _tile8(bz), _tile8(br), _tile8(bn)
    b_out2 = b_out.reshape(1, 1)

    full = lambda shape: pl.BlockSpec(shape, lambda i: tuple(0 for _ in shape))

    h = pl.pallas_call(
        _proj_body,
        grid=(1,),
        in_specs=[
            full((np_, PK * f)),
            full((PK * f, PK * H)),
            full((1, PK * H)),
        ],
        out_specs=full((np_, PK * H)),
        out_shape=jax.ShapeDtypeStruct((np_, PK * H), F32),
    )(nf_p, w_inb, b_int)

    sc_gather = _make_sc_gather(n, e_pad, epw, nchunk)
    sc_scatter = _make_sc_scatter(m_pad, e_pad, epw, nchunk)

    msgs_call = pl.pallas_call(
        _msgs_body,
        grid=(neb,),
        in_specs=[
            pl.BlockSpec((tep, PK * de), lambda i: (i, 0)),
            pl.BlockSpec((tep, PK * H), lambda i: (i, 0)),
            full((PK * de, PK * H)),
            full((1, PK * H)),
            full((PK * H, PK * HH)),
            full((1, PK * HH)),
            full((PK * H, PK * HH)),
            full((PK * HH, PK * H)),
        ],
        out_specs=pl.BlockSpec((tep, PK * H), lambda i: (i, 0)),
        out_shape=jax.ShapeDtypeStruct((ep_, PK * H), F32),
    )

    gru_in_specs = [
        pl.BlockSpec((1, mp_, PK * H), lambda i: (0, 0, 0)),
        pl.BlockSpec((1, mp_, PK * H), lambda i: (1, 0, 0)),
        full((np_, PK * H)),
        full((PK * H, PK * H)), full((PK * H, PK * H)), full((1, PK * H)),
        full((PK * H, PK * H)), full((PK * H, PK * H)), full((1, PK * H)),
        full((PK * H, PK * H)), full((PK * H, PK * H)), full((1, PK * H)),
    ]
    gru_call = pl.pallas_call(
        _gru_body,
        grid=(1,),
        in_specs=gru_in_specs,
        out_specs=full((np_, PK * H)),
        out_shape=jax.ShapeDtypeStruct((np_, PK * H), F32),
    )
    gru_out_call = pl.pallas_call(
        _gru_out_body,
        grid=(1,),
        in_specs=gru_in_specs + [full((PK * H, PK))],
        out_specs=full((np_, PK)),
        out_shape=jax.ShapeDtypeStruct((np_, PK), F32),
    )
    readout_call = pl.pallas_call(
        _readout_body,
        grid=(nb,),
        in_specs=[
            pl.BlockSpec((1, 1, tn), lambda i: (i, 0, 0)),
            pl.BlockSpec((1, 1, tn), lambda i: (i, 0, 0)),
            full((1, 1)),
        ],
        out_specs=pl.BlockSpec((NG, 1), lambda i: (0, 0)),
        out_shape=jax.ShapeDtypeStruct((NG, 1), F32),
        scratch_shapes=[pltpu.VMEM((NG, 1), F32)],
    )

    o_p = None
    for t in range(t_rounds):
        hs = sc_gather(h.reshape(n, H), esrc_p)
        msgs = msgs_call(ef_p, hs.reshape(ep_, PK * H),
                         w1b, b1t, w2b, b2t, rb, sb)
        m2 = sc_scatter(msgs.reshape(e_pad, H), etgt_p, zeros_m)
        m2p = m2.reshape(NC, mp_, PK * H)
        gru_args = (m2p, m2p, h, wzb, uzb, bzt, wrb, urb, brt, wnb, unb, bnt)
        if t < t_rounds - 1:
            h = gru_call(*gru_args)
        else:
            o_p = gru_out_call(*gru_args, wob)
    o3 = o_p.reshape(nb, 1, tn)
    return readout_call(o3, batch3, b_out2)


# speedup vs baseline: 5.0478x; 2.8983x over previous
import functools

import jax
import jax.numpy as jnp
from jax import lax
from jax.experimental import pallas as pl
from jax.experimental.pallas import tpu as pltpu
from jax.experimental.pallas import tpu_sc as plsc

F32 = jnp.float32
HIGH = lax.Precision.HIGHEST

H = 16
HH = H * H
PK = 8
NC = 2
NS = 16
NW = NC * NS
CH = 128
NG = 64


def _dot(a, b):
    return jnp.dot(a, b, preferred_element_type=F32, precision=HIGH)


def _kron8(w):
    return jnp.kron(jnp.eye(PK, dtype=w.dtype), w)


def _tile8(b):
    return jnp.tile(b.reshape(1, -1), (1, PK)).reshape(1, -1)


def _proj_body(nf, w, b, out):
    out[...] = _dot(nf[...], w[...]) + b[...]


def _dotd(a, b):
    return jnp.dot(a, b, preferred_element_type=F32)


def _msgs_body(ef, hs, w1b, b1t, w2bp, b2tp, sbp, out):
    eh = jnp.maximum(_dotd(ef[...], w1b[...]) + b1t[...], 0.0)
    a = _dotd(eh, w2bp[...]) + b2tp[...]
    h_ = hs[...]
    hrep = jnp.concatenate([h_] * H, axis=1)
    prod = a * hrep
    p_hi = prod.astype(jnp.bfloat16).astype(F32)
    p_lo = prod - p_hi
    out[...] = _dotd(p_hi, sbp[...]) + _dotd(p_lo, sbp[...])


def _gru_core(ma, mb, h, wz, uz, bz, wr, ur, br, wn, un, bn):
    m = ma + mb
    z = jax.nn.sigmoid(_dot(m, wz) + _dot(h, uz) + bz)
    r = jax.nn.sigmoid(_dot(m, wr) + _dot(h, ur) + br)
    n = jnp.tanh(_dot(m, wn) + r * _dot(h, un) + bn)
    return (1.0 - z) * n + z * h


def _gru_body(ma, mb, h, wz, uz, bz, wr, ur, br, wn, un, bn, out):
    np_ = h.shape[0]
    out[...] = _gru_core(ma[0, :np_], mb[0, :np_], h[...],
                         wz[...], uz[...], bz[...],
                         wr[...], ur[...], br[...], wn[...], un[...], bn[...])


def _gru_out_body(ma, mb, h, wz, uz, bz, wr, ur, br, wn, un, bn, wob, oout):
    np_ = h.shape[0]
    hn = _gru_core(ma[0, :np_], mb[0, :np_], h[...],
                   wz[...], uz[...], bz[...],
                   wr[...], ur[...], br[...], wn[...], un[...], bn[...])
    oout[...] = _dot(hn, wob[...])


def _readout_body(o3, b3, bo, g, acc):
    i = pl.program_id(0)
    nb = pl.num_programs(0)
    tn = o3.shape[2]
    rows = lax.broadcasted_iota(jnp.int32, (NG, tn), 0)
    oh = (b3[0] == rows).astype(F32)
    contrib = jnp.sum(oh * o3[0], axis=1, keepdims=True)
    cnt = jnp.sum(oh, axis=1, keepdims=True)

    @pl.when(i == 0)
    def _():
        acc[...] = jnp.zeros_like(acc)

    acc[...] += contrib + cnt * bo[0, 0]

    @pl.when(i == nb - 1)
    def _():
        g[...] = acc[...]


def _make_sc_gather(n_nodes, e_pad, epw, nchunk):
    mesh = plsc.VectorSubcoreMesh(core_axis_name="c", subcore_axis_name="s",
                                  num_cores=NC, num_subcores=NS)

    @functools.partial(
        pl.kernel,
        mesh=mesh,
        out_type=jax.ShapeDtypeStruct((e_pad, H), F32),
        scratch_types=[
            pltpu.VMEM((nchunk, CH), jnp.int32),
            pltpu.VMEM((epw, H), F32),
            pltpu.SemaphoreType.DMA,
        ],
        compiler_params=pltpu.CompilerParams(use_tc_tiling_on_sc=False),
    )
    def sc_gather(h_hbm, idx_hbm, out_hbm, idx_v, rows_v, sem):
        c = lax.axis_index("c")
        s = lax.axis_index("s")
        wid = s * NC + c
        pltpu.sync_copy(idx_hbm.at[wid], idx_v)

        def body(j, carry):
            pltpu.async_copy(
                h_hbm.at[idx_v.at[j]],
                rows_v.at[pl.ds(j * CH, CH), :],
                sem,
            ).wait()
            return carry

        lax.fori_loop(0, nchunk, body, 0)
        pltpu.sync_copy(rows_v, out_hbm.at[pl.ds(wid * epw, epw), :])

    return sc_gather


def _make_sc_scatter(m_pad, e_pad, epw, nchunk):
    mesh = plsc.VectorSubcoreMesh(core_axis_name="c", subcore_axis_name="s",
                                  num_cores=NC, num_subcores=NS)
    rpt = m_pad // NS

    @functools.partial(
        pl.kernel,
        mesh=mesh,
        out_type=jax.ShapeDtypeStruct((NC, m_pad, H), F32),
        scratch_types=[
            pltpu.VMEM((nchunk, CH), jnp.int32),
            pltpu.VMEM((epw, H), F32),
            pltpu.VMEM_SHARED((m_pad, H), F32),
        ],
        compiler_params=pltpu.CompilerParams(use_tc_tiling_on_sc=False),
    )
    def sc_scatter(msgs_hbm, idx_hbm, zeros_hbm, out_hbm, idx_v, msg_v, macc):
        c = lax.axis_index("c")
        s = lax.axis_index("s")
        wid = s * NC + c

        @pl.when(s == 0)
        def _():
            pltpu.sync_copy(zeros_hbm, macc)

        pltpu.sync_copy(idx_hbm.at[wid], idx_v)
        pltpu.sync_copy(msgs_hbm.at[pl.ds(wid * epw, epw), :], msg_v)
        plsc.subcore_barrier()

        def body(j, carry):
            pltpu.sync_copy(
                msg_v.at[pl.ds(j * CH, CH), :],
                macc.at[idx_v.at[j]],
                add=True,
            )
            return carry

        lax.fori_loop(0, nchunk, body, 0)
        plsc.subcore_barrier()

        pltpu.sync_copy(macc.at[pl.ds(s * rpt, rpt), :],
                        msg_v.at[pl.ds(0, rpt), :])
        pltpu.sync_copy(msg_v.at[pl.ds(0, rpt), :],
                        out_hbm.at[c, pl.ds(s * rpt, rpt), :])

    return sc_scatter


def kernel(node_features, edge_features, Esrc, Etgt, batch, W_in, b_in,
           ee_W1, ee_b1, ee_W2, ee_b2, Wz, Uz, bz, Wr, Ur, br, Wn, Un, bn,
           W_out, b_out):
    n, f = node_features.shape
    e, de = edge_features.shape
    t_rounds = 3

    e_pad = ((e + NW * CH - 1) // (NW * CH)) * (NW * CH)
    epw = e_pad // NW
    nchunk = epw // CH
    m_pad = ((n + 1 + NS * PK - 1) // (NS * PK)) * (NS * PK)
    np_ = n // PK
    mp_ = m_pad // PK
    ep_ = e_pad // PK
    tn = 2000
    nb = n // tn
    tnp = np_ // nb
    tep = 256
    neb = ep_ // tep

    ef_p = jnp.pad(edge_features.reshape(e // PK, PK * de),
                   ((0, ep_ - e // PK), (0, 0)))
    esrc_p = jnp.pad(Esrc, (0, e_pad - e)).reshape(NW, nchunk, CH)
    etgt_p = jnp.pad(Etgt, (0, e_pad - e), constant_values=n).reshape(
        NW, nchunk, CH)
    zeros_m = jnp.zeros((m_pad, H), F32)
    batch3 = batch.reshape(nb, 1, tn)
    nf_p = node_features.reshape(np_, PK * f)

    w_inb = _kron8(W_in)
    w1b = _kron8(ee_W1)
    w2b = _kron8(ee_W2)
    ll = jnp.arange(PK * HH)
    tt, qq = ll // (PK * H), ll % (PK * H)
    c_orig = HH * (qq // H) + H * tt + qq % H
    w2bp = w2b[:, c_orig]
    colidx = H * (qq // H) + tt
    sbp = (colidx[:, None] == jnp.arange(PK * H)[None, :]).astype(F32)
    uzb, urb, unb = _kron8(Uz), _kron8(Ur), _kron8(Un)
    wzb, wrb, wnb = _kron8(Wz), _kron8(Wr), _kron8(Wn)
    wob = _kron8(W_out)
    b_int = _tile8(b_in)
    b1t = _tile8(ee_b1)
    b2tp = _tile8(ee_b2)[:, c_orig]
    bzt, brt, bnt = _tile8(bz), _tile8(br), _tile8(bn)
    b_out2 = b_out.reshape(1, 1)

    full = lambda shape: pl.BlockSpec(shape, lambda i: tuple(0 for _ in shape))

    h = pl.pallas_call(
        _proj_body,
        grid=(1,),
        in_specs=[
            full((np_, PK * f)),
            full((PK * f, PK * H)),
            full((1, PK * H)),
        ],
        out_specs=full((np_, PK * H)),
        out_shape=jax.ShapeDtypeStruct((np_, PK * H), F32),
    )(nf_p, w_inb, b_int)

    sc_gather = _make_sc_gather(n, e_pad, epw, nchunk)
    sc_scatter = _make_sc_scatter(m_pad, e_pad, epw, nchunk)

    msgs_call = pl.pallas_call(
        _msgs_body,
        grid=(neb,),
        in_specs=[
            pl.BlockSpec((tep, PK * de), lambda i: (i, 0)),
            pl.BlockSpec((tep, PK * H), lambda i: (i, 0)),
            full((PK * de, PK * H)),
            full((1, PK * H)),
            full((PK * H, PK * HH)),
            full((1, PK * HH)),
            full((PK * HH, PK * H)),
        ],
        out_specs=pl.BlockSpec((tep, PK * H), lambda i: (i, 0)),
        out_shape=jax.ShapeDtypeStruct((ep_, PK * H), F32),
    )

    gru_in_specs = [
        pl.BlockSpec((1, mp_, PK * H), lambda i: (0, 0, 0)),
        pl.BlockSpec((1, mp_, PK * H), lambda i: (1, 0, 0)),
        full((np_, PK * H)),
        full((PK * H, PK * H)), full((PK * H, PK * H)), full((1, PK * H)),
        full((PK * H, PK * H)), full((PK * H, PK * H)), full((1, PK * H)),
        full((PK * H, PK * H)), full((PK * H, PK * H)), full((1, PK * H)),
    ]
    gru_call = pl.pallas_call(
        _gru_body,
        grid=(1,),
        in_specs=gru_in_specs,
        out_specs=full((np_, PK * H)),
        out_shape=jax.ShapeDtypeStruct((np_, PK * H), F32),
    )
    gru_out_call = pl.pallas_call(
        _gru_out_body,
        grid=(1,),
        in_specs=gru_in_specs + [full((PK * H, PK))],
        out_specs=full((np_, PK)),
        out_shape=jax.ShapeDtypeStruct((np_, PK), F32),
    )
    readout_call = pl.pallas_call(
        _readout_body,
        grid=(nb,),
        in_specs=[
            pl.BlockSpec((1, 1, tn), lambda i: (i, 0, 0)),
            pl.BlockSpec((1, 1, tn), lambda i: (i, 0, 0)),
            full((1, 1)),
        ],
        out_specs=pl.BlockSpec((NG, 1), lambda i: (0, 0)),
        out_shape=jax.ShapeDtypeStruct((NG, 1), F32),
        scratch_shapes=[pltpu.VMEM((NG, 1), F32)],
    )

    o_p = None
    for t in range(t_rounds):
        hs = sc_gather(h.reshape(n, H), esrc_p)
        msgs = msgs_call(ef_p, hs.reshape(ep_, PK * H),
                         w1b, b1t, w2bp, b2tp, sbp)
        m2 = sc_scatter(msgs.reshape(e_pad, H), etgt_p, zeros_m)
        m2p = m2.reshape(NC, mp_, PK * H)
        gru_args = (m2p, m2p, h, wzb, uzb, bzt, wrb, urb, brt, wnb, unb, bnt)
        if t < t_rounds - 1:
            h = gru_call(*gru_args)
        else:
            o_p = gru_out_call(*gru_args, wob)
    o3 = o_p.reshape(nb, 1, tn)
    return readout_call(o3, batch3, b_out2)


# speedup vs baseline: 6.6714x; 1.3216x over previous
import functools

import jax
import jax.numpy as jnp
from jax import lax
from jax.experimental import pallas as pl
from jax.experimental.pallas import tpu as pltpu
from jax.experimental.pallas import tpu_sc as plsc

F32 = jnp.float32
HIGH = lax.Precision.HIGHEST

H = 16
HH = H * H
PK = 8
NC = 2
NS = 16
NW = NC * NS
CH = 128
NG = 64


def _dot(a, b):
    return jnp.dot(a, b, preferred_element_type=F32)


def _kron8(w):
    return jnp.kron(jnp.eye(PK, dtype=w.dtype), w)


def _tile8(b):
    return jnp.tile(b.reshape(1, -1), (1, PK)).reshape(1, -1)


def _proj_body(nf, w, b, out):
    x = nf[...]
    np_ = out.shape[0]
    parts = [_dot(x[np_ * j:np_ * (j + 1), :], w[...]) for j in range(PK)]
    out[...] = jnp.concatenate(parts, axis=1) + b[...]


def _dotd(a, b):
    return jnp.dot(a, b, preferred_element_type=F32)


def _msgs_body(ef, hs, w1b, b1t, w2bp, b2tp, sbp, out):
    eh = jnp.maximum(_dotd(ef[...], w1b[...]) + b1t[...], 0.0)
    a = _dotd(eh, w2bp[...]) + b2tp[...]
    h_ = hs[...]
    hrep = jnp.concatenate([h_] * H, axis=1)
    out[...] = _dotd(a * hrep, sbp[...])


def _gru_core(ma, mb, h, wz, uz, bz, wr, ur, br, wn, un, bn):
    m = ma + mb
    z = jax.nn.sigmoid(_dot(m, wz) + _dot(h, uz) + bz)
    r = jax.nn.sigmoid(_dot(m, wr) + _dot(h, ur) + br)
    n = jnp.tanh(_dot(m, wn) + r * _dot(h, un) + bn)
    return (1.0 - z) * n + z * h


def _gru_body(ma, mb, h, wz, uz, bz, wr, ur, br, wn, un, bn, out):
    np_ = h.shape[0]
    out[...] = _gru_core(ma[0, :np_], mb[0, :np_], h[...],
                         wz[...], uz[...], bz[...],
                         wr[...], ur[...], br[...], wn[...], un[...], bn[...])


def _gru_out_body(ma, mb, h, wz, uz, bz, wr, ur, br, wn, un, bn, wob, oout):
    np_ = h.shape[0]
    hn = _gru_core(ma[0, :np_], mb[0, :np_], h[...],
                   wz[...], uz[...], bz[...],
                   wr[...], ur[...], br[...], wn[...], un[...], bn[...])
    oout[...] = _dot(hn, wob[...])


def _readout_body(o3, b3, bo, g, acc):
    i = pl.program_id(0)
    nb = pl.num_programs(0)
    tn = o3.shape[2]
    rows = lax.broadcasted_iota(jnp.int32, (NG, tn), 0)
    oh = (b3[0] == rows).astype(F32)
    contrib = jnp.sum(oh * o3[0], axis=1, keepdims=True)
    cnt = jnp.sum(oh, axis=1, keepdims=True)

    @pl.when(i == 0)
    def _():
        acc[...] = jnp.zeros_like(acc)

    acc[...] += contrib + cnt * bo[0, 0]

    @pl.when(i == nb - 1)
    def _():
        g[...] = acc[...]


def _make_sc_gather(n_nodes, e_pad, epw, nchunk):
    mesh = plsc.VectorSubcoreMesh(core_axis_name="c", subcore_axis_name="s",
                                  num_cores=NC, num_subcores=NS)

    @functools.partial(
        pl.kernel,
        mesh=mesh,
        out_type=jax.ShapeDtypeStruct((e_pad, H), F32),
        scratch_types=[
            pltpu.VMEM((nchunk, CH), jnp.int32),
            pltpu.VMEM((epw, H), F32),
            pltpu.SemaphoreType.DMA,
            pltpu.SemaphoreType.DMA,
        ],
        compiler_params=pltpu.CompilerParams(use_tc_tiling_on_sc=False),
    )
    def sc_gather(h_hbm, idx_hbm, out_hbm, idx_v, rows_v, sem0, sem1):
        c = lax.axis_index("c")
        s = lax.axis_index("s")
        wid = s * NC + c
        pltpu.sync_copy(idx_hbm.at[wid], idx_v)

        def fire(j, sem):
            pltpu.async_copy(
                h_hbm.at[idx_v.at[j]],
                rows_v.at[pl.ds(j * CH, CH), :],
                sem,
            )

        def drain(sem):
            pltpu.make_async_copy(
                h_hbm.at[idx_v.at[0]],
                rows_v.at[pl.ds(0, CH), :],
                sem,
            ).wait()

        fire(0, sem0)
        fire(1, sem1)

        def body(jj, carry):
            j = jj * 2
            drain(sem0)
            fire(j + 2, sem0)
            drain(sem1)
            fire(j + 3, sem1)
            return carry

        lax.fori_loop(0, nchunk // 2 - 1, body, 0)
        drain(sem0)
        drain(sem1)
        pltpu.sync_copy(rows_v, out_hbm.at[pl.ds(wid * epw, epw), :])

    return sc_gather


def _make_sc_scatter(m_pad, e_pad, epw, nchunk):
    mesh = plsc.VectorSubcoreMesh(core_axis_name="c", subcore_axis_name="s",
                                  num_cores=NC, num_subcores=NS)
    rpt = m_pad // NS

    @functools.partial(
        pl.kernel,
        mesh=mesh,
        out_type=jax.ShapeDtypeStruct((NC, m_pad, H), F32),
        scratch_types=[
            pltpu.VMEM((nchunk, CH), jnp.int32),
            pltpu.VMEM((epw, H), F32),
            pltpu.VMEM_SHARED((m_pad, H), F32),
        ],
        compiler_params=pltpu.CompilerParams(use_tc_tiling_on_sc=False),
    )
    def sc_scatter(msgs_hbm, idx_hbm, zeros_hbm, out_hbm, idx_v, msg_v, macc):
        c = lax.axis_index("c")
        s = lax.axis_index("s")
        wid = s * NC + c

        @pl.when(s == 0)
        def _():
            pltpu.sync_copy(zeros_hbm, macc)

        pltpu.sync_copy(idx_hbm.at[wid], idx_v)
        pltpu.sync_copy(msgs_hbm.at[pl.ds(wid * epw, epw), :], msg_v)
        plsc.subcore_barrier()

        def body(j, carry):
            pltpu.sync_copy(
                msg_v.at[pl.ds(j * CH, CH), :],
                macc.at[idx_v.at[j]],
                add=True,
            )
            return carry

        lax.fori_loop(0, nchunk, body, 0)
        plsc.subcore_barrier()

        pltpu.sync_copy(macc.at[pl.ds(s * rpt, rpt), :],
                        msg_v.at[pl.ds(0, rpt), :])
        pltpu.sync_copy(msg_v.at[pl.ds(0, rpt), :],
                        out_hbm.at[c, pl.ds(s * rpt, rpt), :])

    return sc_scatter


def kernel(node_features, edge_features, Esrc, Etgt, batch, W_in, b_in,
           ee_W1, ee_b1, ee_W2, ee_b2, Wz, Uz, bz, Wr, Ur, br, Wn, Un, bn,
           W_out, b_out):
    n, f = node_features.shape
    e, de = edge_features.shape
    t_rounds = 3

    e_pad = ((e + NW * CH - 1) // (NW * CH)) * (NW * CH)
    epw = e_pad // NW
    nchunk = epw // CH
    m_pad = ((n + 1 + NS * PK - 1) // (NS * PK)) * (NS * PK)
    np_ = n // PK
    mp_ = m_pad // PK
    ep_ = e_pad // PK
    tn = 2000
    nb = n // tn
    tnp = np_ // nb
    tep = 256
    neb = ep_ // tep

    del tnp
    ef_p = jnp.pad(edge_features.reshape(e // PK, PK * de),
                   ((0, ep_ - e // PK), (0, 0)))
    esrc_r = PK * (Esrc % np_) + Esrc // np_
    etgt_r = PK * (Etgt % np_) + Etgt // np_
    esrc_p = jnp.pad(esrc_r, (0, e_pad - e)).reshape(NW, nchunk, CH)
    etgt_p = jnp.pad(etgt_r, (0, e_pad - e), constant_values=n).reshape(
        NW, nchunk, CH)
    zeros_m = jnp.zeros((m_pad, H), F32)
    pp = jnp.arange(n)
    batch3 = batch[(pp // PK) + np_ * (pp % PK)].reshape(nb, 1, tn)

    w1b = _kron8(ee_W1)
    w2b = _kron8(ee_W2)
    ll = jnp.arange(PK * HH)
    tt, qq = ll // (PK * H), ll % (PK * H)
    c_orig = HH * (qq // H) + H * tt + qq % H
    w2bp = w2b[:, c_orig]
    colidx = H * (qq // H) + tt
    sbp = (colidx[:, None] == jnp.arange(PK * H)[None, :]).astype(F32)
    uzb, urb, unb = _kron8(Uz), _kron8(Ur), _kron8(Un)
    wzb, wrb, wnb = _kron8(Wz), _kron8(Wr), _kron8(Wn)
    wob = _kron8(W_out)
    b_int = _tile8(b_in)
    b1t = _tile8(ee_b1)
    b2tp = _tile8(ee_b2)[:, c_orig]
    bzt, brt, bnt = _tile8(bz), _tile8(br), _tile8(bn)
    b_out2 = b_out.reshape(1, 1)

    full = lambda shape: pl.BlockSpec(shape, lambda i: tuple(0 for _ in shape))

    h = pl.pallas_call(
        _proj_body,
        grid=(1,),
        in_specs=[
            full((n, f)),
            full((f, H)),
            full((1, PK * H)),
        ],
        out_specs=full((np_, PK * H)),
        out_shape=jax.ShapeDtypeStruct((np_, PK * H), F32),
    )(node_features, W_in, b_int)

    sc_gather = _make_sc_gather(n, e_pad, epw, nchunk)
    sc_scatter = _make_sc_scatter(m_pad, e_pad, epw, nchunk)

    msgs_call = pl.pallas_call(
        _msgs_body,
        grid=(neb,),
        in_specs=[
            pl.BlockSpec((tep, PK * de), lambda i: (i, 0)),
            pl.BlockSpec((tep, PK * H), lambda i: (i, 0)),
            full((PK * de, PK * H)),
            full((1, PK * H)),
            full((PK * H, PK * HH)),
            full((1, PK * HH)),
            full((PK * HH, PK * H)),
        ],
        out_specs=pl.BlockSpec((tep, PK * H), lambda i: (i, 0)),
        out_shape=jax.ShapeDtypeStruct((ep_, PK * H), F32),
    )

    gru_in_specs = [
        pl.BlockSpec((1, mp_, PK * H), lambda i: (0, 0, 0)),
        pl.BlockSpec((1, mp_, PK * H), lambda i: (1, 0, 0)),
        full((np_, PK * H)),
        full((PK * H, PK * H)), full((PK * H, PK * H)), full((1, PK * H)),
        full((PK * H, PK * H)), full((PK * H, PK * H)), full((1, PK * H)),
        full((PK * H, PK * H)), full((PK * H, PK * H)), full((1, PK * H)),
    ]
    gru_call = pl.pallas_call(
        _gru_body,
        grid=(1,),
        in_specs=gru_in_specs,
        out_specs=full((np_, PK * H)),
        out_shape=jax.ShapeDtypeStruct((np_, PK * H), F32),
    )
    gru_out_call = pl.pallas_call(
        _gru_out_body,
        grid=(1,),
        in_specs=gru_in_specs + [full((PK * H, PK))],
        out_specs=full((np_, PK)),
        out_shape=jax.ShapeDtypeStruct((np_, PK), F32),
    )
    readout_call = pl.pallas_call(
        _readout_body,
        grid=(nb,),
        in_specs=[
            pl.BlockSpec((1, 1, tn), lambda i: (i, 0, 0)),
            pl.BlockSpec((1, 1, tn), lambda i: (i, 0, 0)),
            full((1, 1)),
        ],
        out_specs=pl.BlockSpec((NG, 1), lambda i: (0, 0)),
        out_shape=jax.ShapeDtypeStruct((NG, 1), F32),
        scratch_shapes=[pltpu.VMEM((NG, 1), F32)],
    )

    o_p = None
    for t in range(t_rounds):
        hs = sc_gather(h.reshape(n, H), esrc_p)
        msgs = msgs_call(ef_p, hs.reshape(ep_, PK * H),
                         w1b, b1t, w2bp, b2tp, sbp)
        m2 = sc_scatter(msgs.reshape(e_pad, H), etgt_p, zeros_m)
        m2p = m2.reshape(NC, mp_, PK * H)
        gru_args = (m2p, m2p, h, wzb, uzb, bzt, wrb, urb, brt, wnb, unb, bnt)
        if t < t_rounds - 1:
            h = gru_call(*gru_args)
        else:
            o_p = gru_out_call(*gru_args, wob)
    o3 = o_p.reshape(nb, 1, tn)
    return readout_call(o3, batch3, b_out2)
